# Initial kernel scaffold; baseline (speedup 1.0000x reference)
import functools

import jax
import jax.numpy as jnp
from jax import lax
from jax.experimental import pallas as pl
from jax.experimental.pallas import tpu as pltpu
from jax.experimental.pallas import tpu_sc as plsc

N = 50000
D = 64
T = 20
NC = 2
NS = 16
LN_EPS = 1e-5

HALF = 25000
SP_TPR = 1568
SP_ROWS = NS * SP_TPR
TRASH = 25024
OUT_TPR = 1564

DEG_TPR = 3128
DEG_ROWS = NS * DEG_TPR

CH_E = 5120
CH_J = CH_E // 128


def _sc_mesh():
    return plsc.VectorSubcoreMesh(
        core_axis_name="c", subcore_axis_name="s", num_cores=NC, num_subcores=NS
    )


def _make_degree_kernel(e_pad):
    per_tile = e_pad // (NC * NS)
    n_j = per_tile // 128

    @functools.partial(
        pl.kernel,
        out_type=jax.ShapeDtypeStruct((NC * DEG_ROWS, 1), jnp.float32),
        mesh=_sc_mesh(),
        scratch_types=[
            pltpu.VMEM((per_tile,), jnp.int32),
            pltpu.VMEM((1, 128), jnp.int32),
            pltpu.VMEM((128, 1), jnp.float32),
            pltpu.VMEM((DEG_TPR, 1), jnp.float32),
            pltpu.VMEM_SHARED((DEG_ROWS, 1), jnp.float32),
        ],
    )
    def deg_kernel(dst_hbm, ones_hbm, zeros_hbm, out_hbm, dbuf, stg, ones_v, z_v, hist):
        cc = lax.axis_index("c")
        ss = lax.axis_index("s")
        pltpu.sync_copy(ones_hbm, ones_v)
        pltpu.sync_copy(zeros_hbm, z_v)
        pltpu.sync_copy(z_v, hist.at[pl.ds(ss * DEG_TPR, DEG_TPR)])
        plsc.subcore_barrier()

        ebase = (cc * NS + ss) * per_tile
        pltpu.sync_copy(dst_hbm.at[pl.ds(ebase, per_tile)], dbuf)

        def body(j, carry):
            def fix(u, c2):
                stg[0, pl.ds(u * 16, 16)] = dbuf[pl.ds(j * 128 + u * 16, 16)]
                return c2

            lax.fori_loop(0, 8, fix, 0)
            pltpu.sync_copy(ones_v, hist.at[stg.at[0]], add=True)
            return carry

        lax.fori_loop(0, n_j, body, 0)
        plsc.subcore_barrier()
        pltpu.sync_copy(
            hist.at[pl.ds(ss * DEG_TPR, DEG_TPR)],
            out_hbm.at[pl.ds(cc * DEG_ROWS + ss * DEG_TPR, DEG_TPR)],
        )

    return deg_kernel


def _make_scatter_kernel(e_pad):
    per_tile = e_pad // NS
    n_chunks = per_tile // CH_E

    @functools.partial(
        pl.kernel,
        out_type=jax.ShapeDtypeStruct((NC * NS * OUT_TPR, D), jnp.float32),
        mesh=_sc_mesh(),
        scratch_types=[
            pltpu.VMEM((CH_E,), jnp.int32),
            pltpu.VMEM((CH_E,), jnp.int32),
            pltpu.VMEM((1, 128), jnp.int32),
            pltpu.VMEM((128, D), jnp.float32),
            pltpu.VMEM_SHARED((SP_ROWS, D), jnp.float32),
        ],
    )
    def scat_kernel(src_hbm, dst_hbm, y_hbm, zeros_hbm, out_hbm,
                    sbuf, dbuf, stg, rows, acc):
        cc = lax.axis_index("c")
        ss = lax.axis_index("s")
        pltpu.sync_copy(zeros_hbm, acc.at[pl.ds(ss * SP_TPR, SP_TPR)])
        plsc.subcore_barrier()

        base = cc * HALF

        def chunk(ci, carry):
            ebase = ss * per_tile + ci * CH_E
            pltpu.sync_copy(src_hbm.at[pl.ds(ebase, CH_E)], sbuf)
            pltpu.sync_copy(dst_hbm.at[pl.ds(ebase, CH_E)], dbuf)

            def stream(j, c2):
                def fix(u, c3):
                    v = dbuf[pl.ds(j * 128 + u * 16, 16)]
                    loc = v - base
                    ok = (loc >= 0) & (loc < HALF)
                    stg[0, pl.ds(u * 16, 16)] = jnp.where(ok, loc, TRASH)
                    return c3

                lax.fori_loop(0, 8, fix, 0)
                pltpu.sync_copy(y_hbm.at[sbuf.at[pl.ds(j * 128, 128)]], rows)
                pltpu.sync_copy(rows, acc.at[stg.at[0]], add=True)
                return c2

            lax.fori_loop(0, CH_J, stream, 0)
            return carry

        lax.fori_loop(0, n_chunks, chunk, 0)
        plsc.subcore_barrier()
        pltpu.sync_copy(
            acc.at[pl.ds(ss * OUT_TPR, OUT_TPR)],
            out_hbm.at[pl.ds((cc * NS + ss) * OUT_TPR, OUT_TPR)],
        )

    return scat_kernel


BN = 2000


def _ln_rows(x, g, b):
    m = jnp.mean(x, axis=-1, keepdims=True)
    v = jnp.mean((x - m) ** 2, axis=-1, keepdims=True)
    return (x - m) * lax.rsqrt(v + LN_EPS) * g + b


def _dense_a_body(bw_ref, gu_ref, mu_ref, gf_ref, mc_ref, rid_ref, d0_ref, d1_ref,
                  remb_ref, wgf_ref, wmc_ref, bs_ref, wih_ref, bih_ref,
                  whh_ref, bhh_ref, lng_ref, lnb_ref, w1t_ref,
                  y1_ref, dinv_ref):
    gf = jnp.log1p(jnp.maximum(gf_ref[...], 0.0))
    mc = jnp.log1p(jnp.maximum(mc_ref[...], 0.0))
    rid = rid_ref[...]
    h_static = gf * wgf_ref[...] + mc * wmc_ref[...] + bs_ref[...]
    for k in range(3):
        h_static += jnp.where(rid == k, 1.0, 0.0) * remb_ref[k:k + 1, :]

    bw = jnp.log1p(jnp.maximum(bw_ref[...], 0.0))
    gu = gu_ref[...]
    mu = mu_ref[...]
    wih = wih_ref[...]
    bih = bih_ref[...]
    whh = whh_ref[...]
    bhh = bhh_ref[...]

    h = jnp.zeros((BN, D), jnp.float32)
    for t in range(T):
        gi = (bw[:, t:t + 1] * wih[0:1, :] + gu[:, t:t + 1] * wih[1:2, :]
              + mu[:, t:t + 1] * wih[2:3, :] + bih)
        gh = jnp.dot(h, whh, preferred_element_type=jnp.float32) + bhh
        r = jax.nn.sigmoid(gi[:, :D] + gh[:, :D])
        z = jax.nn.sigmoid(gi[:, D:2 * D] + gh[:, D:2 * D])
        n = jnp.tanh(gi[:, 2 * D:] + r * gh[:, 2 * D:])
        h = (1.0 - z) * n + z * h

    h_dyn = _ln_rows(h, lng_ref[...], lnb_ref[...])
    x0 = jnp.maximum(h_static + h_dyn, 0.0)
    dinv = lax.rsqrt(d0_ref[...] + d1_ref[...] + 1.0)
    y1_ref[...] = jnp.dot(x0, w1t_ref[...], preferred_element_type=jnp.float32) * dinv
    dinv_ref[...] = dinv


def _dense_a(bw, gu, mu, gf, mc, rid, d0, d1, remb, wgf, wmc, bs_row,
             wihT, bih_row, whhT, bhh_row, lng, lnb, w1t):
    nb = N // BN
    row = lambda i: (i, 0)
    full = lambda i: (0, 0)
    spec = lambda shape, im: pl.BlockSpec(shape, im)
    return pl.pallas_call(
        _dense_a_body,
        grid=(nb,),
        in_specs=[
            spec((BN, T), row), spec((BN, T), row), spec((BN, T), row),
            spec((BN, 1), row), spec((BN, 1), row), spec((BN, 1), row),
            spec((BN, 1), row), spec((BN, 1), row),
            spec((3, D), full), spec((1, D), full), spec((1, D), full),
            spec((1, D), full), spec((3, 3 * D), full), spec((1, 3 * D), full),
            spec((D, 3 * D), full), spec((1, 3 * D), full),
            spec((1, D), full), spec((1, D), full), spec((D, D), full),
        ],
        out_specs=[spec((BN, D), row), spec((BN, 1), row)],
        out_shape=[
            jax.ShapeDtypeStruct((N, D), jnp.float32),
            jax.ShapeDtypeStruct((N, 1), jnp.float32),
        ],
    )(bw, gu, mu, gf, mc, rid, d0, d1, remb, wgf, wmc, bs_row,
      wihT, bih_row, whhT, bhh_row, lng, lnb, w1t)


def _dense_b_body(acc_ref, y_ref, dinv_ref, b_ref, lng_ref, lnb_ref, w2t_ref, out_ref):
    dinv = dinv_ref[...]
    o = dinv * (acc_ref[...] + y_ref[...]) + b_ref[...]
    h1 = _ln_rows(jnp.maximum(o, 0.0), lng_ref[...], lnb_ref[...])
    out_ref[...] = jnp.dot(h1, w2t_ref[...], preferred_element_type=jnp.float32) * dinv


def _dense_b(acc1, y1, dinv, b_row, lng, lnb, w2t):
    nb = N // BN
    row = lambda i: (i, 0)
    full = lambda i: (0, 0)
    spec = lambda shape, im: pl.BlockSpec(shape, im)
    return pl.pallas_call(
        _dense_b_body,
        grid=(nb,),
        in_specs=[
            spec((BN, D), row), spec((BN, D), row), spec((BN, 1), row),
            spec((1, D), full), spec((1, D), full), spec((1, D), full),
            spec((D, D), full),
        ],
        out_specs=spec((BN, D), row),
        out_shape=jax.ShapeDtypeStruct((N, D), jnp.float32),
    )(acc1, y1, dinv, b_row, lng, lnb, w2t)


def _dense_c_body(acc_ref, y_ref, dinv_ref, b_ref, lng_ref, lnb_ref, out_ref):
    o = dinv_ref[...] * (acc_ref[...] + y_ref[...]) + b_ref[...]
    out_ref[...] = _ln_rows(o, lng_ref[...], lnb_ref[...])


def _dense_c(acc2, y2, dinv, b_row, lng, lnb):
    nb = N // BN
    row = lambda i: (i, 0)
    full = lambda i: (0, 0)
    spec = lambda shape, im: pl.BlockSpec(shape, im)
    return pl.pallas_call(
        _dense_c_body,
        grid=(nb,),
        in_specs=[
            spec((BN, D), row), spec((BN, D), row), spec((BN, 1), row),
            spec((1, D), full), spec((1, D), full), spec((1, D), full),
        ],
        out_specs=spec((BN, D), row),
        out_shape=jax.ShapeDtypeStruct((N, D), jnp.float32),
    )(acc2, y2, dinv, b_row, lng, lnb)


def kernel(edge_index, gpu_flops, role_id, mem_capacity, bandwidth_seq,
           gpu_util_seq, mem_util_seq, role_emb, Ws, bs, Wih, Whh, bih, bhh,
           lnt_g, lnt_b, W1, b1, W2, b2, ln1_g, ln1_b, ln2_g, ln2_b):
    E = edge_index.shape[1]
    grain = NS * CH_E
    e_pad = ((E + grain - 1) // grain) * grain

    src = edge_index[0]
    dst = edge_index[1]
    src_p = jnp.concatenate([src, jnp.zeros((e_pad - E,), jnp.int32)])
    dst_p = jnp.concatenate([dst, jnp.full((e_pad - E,), N, jnp.int32)])

    ones_col = jnp.ones((128, 1), jnp.float32)
    zeros_deg = jnp.zeros((DEG_TPR, 1), jnp.float32)
    zeros_acc = jnp.zeros((SP_TPR, D), jnp.float32)

    deg_kernel = _make_degree_kernel(e_pad)
    deg_parts = deg_kernel(dst_p, ones_col, zeros_deg)
    d0 = deg_parts[:N]
    d1 = deg_parts[DEG_ROWS:DEG_ROWS + N]

    WsT = Ws.T
    wgf = WsT[0:1, :]
    wmc = WsT[1:2, :]
    remb = role_emb @ WsT[2:10, :]
    bs_row = bs[None, :]
    wihT = Wih.T
    whhT = Whh.T
    bih_row = bih[None, :]
    bhh_row = bhh[None, :]

    y1, dinv = _dense_a(
        bandwidth_seq, gpu_util_seq, mem_util_seq,
        gpu_flops[:, None], mem_capacity[:, None], role_id[:, None],
        d0, d1, remb, wgf, wmc, bs_row, wihT, bih_row, whhT, bhh_row,
        lnt_g[None, :], lnt_b[None, :], W1.T)

    scat_kernel = _make_scatter_kernel(e_pad)

    acc1_raw = scat_kernel(src_p, dst_p, y1, zeros_acc)
    acc1 = jnp.concatenate(
        [acc1_raw[:HALF], acc1_raw[NS * OUT_TPR:NS * OUT_TPR + HALF]], axis=0)

    y2 = _dense_b(acc1, y1, dinv, b1[None, :], ln1_g[None, :], ln1_b[None, :], W2.T)

    acc2_raw = scat_kernel(src_p, dst_p, y2, zeros_acc)
    acc2 = jnp.concatenate(
        [acc2_raw[:HALF], acc2_raw[NS * OUT_TPR:NS * OUT_TPR + HALF]], axis=0)

    return _dense_c(acc2, y2, dinv, b2[None, :], ln2_g[None, :], ln2_b[None, :])


# speedup vs baseline: 6.6029x; 6.6029x over previous
import functools

import jax
import jax.numpy as jnp
from jax import lax
from jax.experimental import pallas as pl
from jax.experimental.pallas import tpu as pltpu
from jax.experimental.pallas import tpu_sc as plsc

N = 50000
D = 64
T = 20
NC = 2
NS = 16
LN_EPS = 1e-5

HALF = 25000
SP_TPR = 1568
SP_ROWS = NS * SP_TPR
TRASH = 25024
OUT_TPR = 1564

DEG_TPR = 3128
DEG_ROWS = NS * DEG_TPR

CH_E = 5120
CH_J = CH_E // 128


def _sc_mesh():
    return plsc.VectorSubcoreMesh(
        core_axis_name="c", subcore_axis_name="s", num_cores=NC, num_subcores=NS
    )


_SC_PARAMS = pltpu.CompilerParams(use_tc_tiling_on_sc=False)


def _make_degree_kernel(e_pad):
    per_tile = e_pad // (NC * NS)
    n_j = per_tile // 128

    @functools.partial(
        pl.kernel,
        out_type=jax.ShapeDtypeStruct((NC * DEG_ROWS, 1), jnp.float32),
        mesh=_sc_mesh(),
        scratch_types=[
            pltpu.VMEM((per_tile,), jnp.int32),
            pltpu.VMEM((1, 128), jnp.int32),
            pltpu.VMEM((128, 1), jnp.float32),
            pltpu.VMEM((DEG_TPR, 1), jnp.float32),
            pltpu.VMEM_SHARED((DEG_ROWS, 1), jnp.float32),
        ],
        compiler_params=_SC_PARAMS,
    )
    def deg_kernel(dst_hbm, ones_hbm, zeros_hbm, out_hbm, dbuf, stg, ones_v, z_v, hist):
        cc = lax.axis_index("c")
        ss = lax.axis_index("s")
        pltpu.sync_copy(ones_hbm, ones_v)
        pltpu.sync_copy(zeros_hbm, z_v)
        pltpu.sync_copy(z_v, hist.at[pl.ds(ss * DEG_TPR, DEG_TPR)])
        plsc.subcore_barrier()

        ebase = (cc * NS + ss) * per_tile
        pltpu.sync_copy(dst_hbm.at[pl.ds(ebase, per_tile)], dbuf)

        def body(j, carry):
            def fix(u, c2):
                stg[0, pl.ds(u * 16, 16)] = dbuf[pl.ds(j * 128 + u * 16, 16)]
                return c2

            lax.fori_loop(0, 8, fix, 0)
            pltpu.sync_copy(ones_v, hist.at[stg.at[0]], add=True)
            return carry

        lax.fori_loop(0, n_j, body, 0)
        plsc.subcore_barrier()
        pltpu.sync_copy(
            hist.at[pl.ds(ss * DEG_TPR, DEG_TPR)],
            out_hbm.at[pl.ds(cc * DEG_ROWS + ss * DEG_TPR, DEG_TPR)],
        )

    return deg_kernel


def _make_scatter_kernel(e_pad):
    per_tile = e_pad // NS
    n_chunks = per_tile // CH_E

    @functools.partial(
        pl.kernel,
        out_type=jax.ShapeDtypeStruct((NC * NS * OUT_TPR, D), jnp.float32),
        mesh=_sc_mesh(),
        scratch_types=[
            pltpu.VMEM((CH_E,), jnp.int32),
            pltpu.VMEM((CH_E,), jnp.int32),
            pltpu.VMEM((1, 128), jnp.int32),
            pltpu.VMEM((128, D), jnp.float32),
            pltpu.VMEM_SHARED((SP_ROWS, D), jnp.float32),
        ],
        compiler_params=_SC_PARAMS,
    )
    def scat_kernel(src_hbm, dst_hbm, y_hbm, zeros_hbm, out_hbm,
                    sbuf, dbuf, stg, rows, acc):
        cc = lax.axis_index("c")
        ss = lax.axis_index("s")
        pltpu.sync_copy(zeros_hbm, acc.at[pl.ds(ss * SP_TPR, SP_TPR)])
        plsc.subcore_barrier()

        base = cc * HALF

        def chunk(ci, carry):
            ebase = ss * per_tile + ci * CH_E
            pltpu.sync_copy(src_hbm.at[pl.ds(ebase, CH_E)], sbuf)
            pltpu.sync_copy(dst_hbm.at[pl.ds(ebase, CH_E)], dbuf)

            def stream(j, c2):
                def fix(u, c3):
                    v = dbuf[pl.ds(j * 128 + u * 16, 16)]
                    loc = v - base
                    ok = (loc >= 0) & (loc < HALF)
                    stg[0, pl.ds(u * 16, 16)] = jnp.where(ok, loc, TRASH)
                    return c3

                lax.fori_loop(0, 8, fix, 0)
                pltpu.sync_copy(y_hbm.at[sbuf.at[pl.ds(j * 128, 128)]], rows)
                pltpu.sync_copy(rows, acc.at[stg.at[0]], add=True)
                return c2

            lax.fori_loop(0, CH_J, stream, 0)
            return carry

        lax.fori_loop(0, n_chunks, chunk, 0)
        plsc.subcore_barrier()
        pltpu.sync_copy(
            acc.at[pl.ds(ss * OUT_TPR, OUT_TPR)],
            out_hbm.at[pl.ds((cc * NS + ss) * OUT_TPR, OUT_TPR)],
        )

    return scat_kernel


BN = 2000


def _ln_rows(x, g, b):
    m = jnp.mean(x, axis=-1, keepdims=True)
    v = jnp.mean((x - m) ** 2, axis=-1, keepdims=True)
    return (x - m) * lax.rsqrt(v + LN_EPS) * g + b


def _dense_a_body(bw_ref, gu_ref, mu_ref, gf_ref, mc_ref, rid_ref, d0_ref, d1_ref,
                  remb_ref, wgf_ref, wmc_ref, bs_ref, wih_ref, bih_ref,
                  whh_ref, bhh_ref, lng_ref, lnb_ref, w1t_ref,
                  y1_ref, dinv_ref):
    gf = jnp.log1p(jnp.maximum(gf_ref[...], 0.0))
    mc = jnp.log1p(jnp.maximum(mc_ref[...], 0.0))
    rid = rid_ref[...]
    h_static = gf * wgf_ref[...] + mc * wmc_ref[...] + bs_ref[...]
    for k in range(3):
        h_static += jnp.where(rid == k, 1.0, 0.0) * remb_ref[k:k + 1, :]

    bw = jnp.log1p(jnp.maximum(bw_ref[...], 0.0))
    gu = gu_ref[...]
    mu = mu_ref[...]
    wih = wih_ref[...]
    bih = bih_ref[...]
    whh = whh_ref[...]
    bhh = bhh_ref[...]

    h = jnp.zeros((BN, D), jnp.float32)
    for t in range(T):
        gi = (bw[:, t:t + 1] * wih[0:1, :] + gu[:, t:t + 1] * wih[1:2, :]
              + mu[:, t:t + 1] * wih[2:3, :] + bih)
        gh = jnp.dot(h, whh, preferred_element_type=jnp.float32) + bhh
        r = jax.nn.sigmoid(gi[:, :D] + gh[:, :D])
        z = jax.nn.sigmoid(gi[:, D:2 * D] + gh[:, D:2 * D])
        n = jnp.tanh(gi[:, 2 * D:] + r * gh[:, 2 * D:])
        h = (1.0 - z) * n + z * h

    h_dyn = _ln_rows(h, lng_ref[...], lnb_ref[...])
    x0 = jnp.maximum(h_static + h_dyn, 0.0)
    dinv = lax.rsqrt(d0_ref[...] + d1_ref[...] + 1.0)
    y1_ref[...] = jnp.dot(x0, w1t_ref[...], preferred_element_type=jnp.float32) * dinv
    dinv_ref[...] = dinv


def _dense_a(bw, gu, mu, gf, mc, rid, d0, d1, remb, wgf, wmc, bs_row,
             wihT, bih_row, whhT, bhh_row, lng, lnb, w1t):
    nb = N // BN
    row = lambda i: (i, 0)
    full = lambda i: (0, 0)
    spec = lambda shape, im: pl.BlockSpec(shape, im)
    return pl.pallas_call(
        _dense_a_body,
        grid=(nb,),
        in_specs=[
            spec((BN, T), row), spec((BN, T), row), spec((BN, T), row),
            spec((BN, 1), row), spec((BN, 1), row), spec((BN, 1), row),
            spec((BN, 1), row), spec((BN, 1), row),
            spec((3, D), full), spec((1, D), full), spec((1, D), full),
            spec((1, D), full), spec((3, 3 * D), full), spec((1, 3 * D), full),
            spec((D, 3 * D), full), spec((1, 3 * D), full),
            spec((1, D), full), spec((1, D), full), spec((D, D), full),
        ],
        out_specs=[spec((BN, D), row), spec((BN, 1), row)],
        out_shape=[
            jax.ShapeDtypeStruct((N, D), jnp.float32),
            jax.ShapeDtypeStruct((N, 1), jnp.float32),
        ],
    )(bw, gu, mu, gf, mc, rid, d0, d1, remb, wgf, wmc, bs_row,
      wihT, bih_row, whhT, bhh_row, lng, lnb, w1t)


def _dense_b_body(acc_ref, y_ref, dinv_ref, b_ref, lng_ref, lnb_ref, w2t_ref, out_ref):
    dinv = dinv_ref[...]
    o = dinv * (acc_ref[...] + y_ref[...]) + b_ref[...]
    h1 = _ln_rows(jnp.maximum(o, 0.0), lng_ref[...], lnb_ref[...])
    out_ref[...] = jnp.dot(h1, w2t_ref[...], preferred_element_type=jnp.float32) * dinv


def _dense_b(acc1, y1, dinv, b_row, lng, lnb, w2t):
    nb = N // BN
    row = lambda i: (i, 0)
    full = lambda i: (0, 0)
    spec = lambda shape, im: pl.BlockSpec(shape, im)
    return pl.pallas_call(
        _dense_b_body,
        grid=(nb,),
        in_specs=[
            spec((BN, D), row), spec((BN, D), row), spec((BN, 1), row),
            spec((1, D), full), spec((1, D), full), spec((1, D), full),
            spec((D, D), full),
        ],
        out_specs=spec((BN, D), row),
        out_shape=jax.ShapeDtypeStruct((N, D), jnp.float32),
    )(acc1, y1, dinv, b_row, lng, lnb, w2t)


def _dense_c_body(acc_ref, y_ref, dinv_ref, b_ref, lng_ref, lnb_ref, out_ref):
    o = dinv_ref[...] * (acc_ref[...] + y_ref[...]) + b_ref[...]
    out_ref[...] = _ln_rows(o, lng_ref[...], lnb_ref[...])


def _dense_c(acc2, y2, dinv, b_row, lng, lnb):
    nb = N // BN
    row = lambda i: (i, 0)
    full = lambda i: (0, 0)
    spec = lambda shape, im: pl.BlockSpec(shape, im)
    return pl.pallas_call(
        _dense_c_body,
        grid=(nb,),
        in_specs=[
            spec((BN, D), row), spec((BN, D), row), spec((BN, 1), row),
            spec((1, D), full), spec((1, D), full), spec((1, D), full),
        ],
        out_specs=spec((BN, D), row),
        out_shape=jax.ShapeDtypeStruct((N, D), jnp.float32),
    )(acc2, y2, dinv, b_row, lng, lnb)


def kernel(edge_index, gpu_flops, role_id, mem_capacity, bandwidth_seq,
           gpu_util_seq, mem_util_seq, role_emb, Ws, bs, Wih, Whh, bih, bhh,
           lnt_g, lnt_b, W1, b1, W2, b2, ln1_g, ln1_b, ln2_g, ln2_b):
    E = edge_index.shape[1]
    grain = NS * CH_E
    e_pad = ((E + grain - 1) // grain) * grain

    src = edge_index[0]
    dst = edge_index[1]
    src_p = jnp.concatenate([src, jnp.zeros((e_pad - E,), jnp.int32)])
    dst_p = jnp.concatenate([dst, jnp.full((e_pad - E,), N, jnp.int32)])

    ones_col = jnp.ones((128, 1), jnp.float32)
    zeros_deg = jnp.zeros((DEG_TPR, 1), jnp.float32)
    zeros_acc = jnp.zeros((SP_TPR, D), jnp.float32)

    deg_kernel = _make_degree_kernel(e_pad)
    deg_parts = deg_kernel(dst_p, ones_col, zeros_deg)
    d0 = deg_parts[:N]
    d1 = deg_parts[DEG_ROWS:DEG_ROWS + N]

    WsT = Ws.T
    wgf = WsT[0:1, :]
    wmc = WsT[1:2, :]
    remb = role_emb @ WsT[2:10, :]
    bs_row = bs[None, :]
    wihT = Wih.T
    whhT = Whh.T
    bih_row = bih[None, :]
    bhh_row = bhh[None, :]

    y1, dinv = _dense_a(
        bandwidth_seq, gpu_util_seq, mem_util_seq,
        gpu_flops[:, None], mem_capacity[:, None], role_id[:, None],
        d0, d1, remb, wgf, wmc, bs_row, wihT, bih_row, whhT, bhh_row,
        lnt_g[None, :], lnt_b[None, :], W1.T)

    scat_kernel = _make_scatter_kernel(e_pad)

    acc1_raw = scat_kernel(src_p, dst_p, y1, zeros_acc)
    acc1 = jnp.concatenate(
        [acc1_raw[:HALF], acc1_raw[NS * OUT_TPR:NS * OUT_TPR + HALF]], axis=0)

    y2 = _dense_b(acc1, y1, dinv, b1[None, :], ln1_g[None, :], ln1_b[None, :], W2.T)

    acc2_raw = scat_kernel(src_p, dst_p, y2, zeros_acc)
    acc2 = jnp.concatenate(
        [acc2_raw[:HALF], acc2_raw[NS * OUT_TPR:NS * OUT_TPR + HALF]], axis=0)

    return _dense_c(acc2, y2, dinv, b2[None, :], ln2_g[None, :], ln2_b[None, :])


# speedup vs baseline: 6.7076x; 1.0159x over previous
import functools

import jax
import jax.numpy as jnp
from jax import lax
from jax.experimental import pallas as pl
from jax.experimental.pallas import tpu as pltpu
from jax.experimental.pallas import tpu_sc as plsc

N = 50000
D = 64
T = 20
NC = 2
NS = 16
LN_EPS = 1e-5

HALF = 25000
OUT_TPR = 1564
SP_TPR = 1564
SP_ROWS = NS * SP_TPR
TRASH = 25000

DEG_TPR = 3128
DEG_ROWS = NS * DEG_TPR

STREAM = 64
MEGA_E = 2560
NJ = MEGA_E // STREAM
NB = 5
NG = NJ // NB


def _sc_mesh():
    return plsc.VectorSubcoreMesh(
        core_axis_name="c", subcore_axis_name="s", num_cores=NC, num_subcores=NS
    )


_SC_PARAMS = pltpu.CompilerParams(use_tc_tiling_on_sc=False)


def _make_degree_kernel(e_pad):
    per_tile = e_pad // (NC * NS)
    n_j = per_tile // 128

    @functools.partial(
        pl.kernel,
        out_type=jax.ShapeDtypeStruct((NC * DEG_ROWS, 1), jnp.float32),
        mesh=_sc_mesh(),
        scratch_types=[
            pltpu.VMEM((per_tile,), jnp.int32),
            pltpu.VMEM((1, 128), jnp.int32),
            pltpu.VMEM((128, 1), jnp.float32),
            pltpu.VMEM((DEG_TPR, 1), jnp.float32),
            pltpu.VMEM_SHARED((DEG_ROWS, 1), jnp.float32),
        ],
        compiler_params=_SC_PARAMS,
    )
    def deg_kernel(dst_hbm, ones_hbm, zeros_hbm, out_hbm, dbuf, stg, ones_v, z_v, hist):
        cc = lax.axis_index("c")
        ss = lax.axis_index("s")
        pltpu.sync_copy(ones_hbm, ones_v)
        pltpu.sync_copy(zeros_hbm, z_v)
        pltpu.sync_copy(z_v, hist.at[pl.ds(ss * DEG_TPR, DEG_TPR)])
        plsc.subcore_barrier()

        ebase = (cc * NS + ss) * per_tile
        pltpu.sync_copy(dst_hbm.at[pl.ds(ebase, per_tile)], dbuf)

        def body(j, carry):
            def fix(u, c2):
                stg[0, pl.ds(u * 16, 16)] = dbuf[pl.ds(j * 128 + u * 16, 16)]
                return c2

            lax.fori_loop(0, 8, fix, 0)
            pltpu.sync_copy(ones_v, hist.at[stg.at[0]], add=True)
            return carry

        lax.fori_loop(0, n_j, body, 0)
        plsc.subcore_barrier()
        pltpu.sync_copy(
            hist.at[pl.ds(ss * DEG_TPR, DEG_TPR)],
            out_hbm.at[pl.ds(cc * DEG_ROWS + ss * DEG_TPR, DEG_TPR)],
        )

    return deg_kernel


def _make_scatter_kernel(e_pad):
    per_tile = e_pad // NS
    n_megas = per_tile // MEGA_E

    @functools.partial(
        pl.kernel,
        out_type=jax.ShapeDtypeStruct((NC * NS * OUT_TPR, D), jnp.float32),
        mesh=_sc_mesh(),
        scratch_types=[
            pltpu.VMEM((MEGA_E,), jnp.int32),
            pltpu.VMEM((MEGA_E,), jnp.int32),
            pltpu.VMEM((NB, STREAM), jnp.int32),
            pltpu.VMEM((NB, STREAM, D), jnp.float32),
            pltpu.SemaphoreType.DMA,
            pltpu.SemaphoreType.DMA,
            pltpu.VMEM_SHARED((SP_ROWS, D), jnp.float32),
        ],
        compiler_params=_SC_PARAMS,
    )
    def scat_kernel(src_hbm, dst_hbm, y_hbm, zeros_hbm, out_hbm,
                    sbuf, dbuf, stg, rows, gsem, ssem, acc):
        cc = lax.axis_index("c")
        ss = lax.axis_index("s")
        pltpu.sync_copy(zeros_hbm, acc.at[pl.ds(ss * SP_TPR, SP_TPR)])
        plsc.subcore_barrier()

        base = cc * HALF

        def fix(jj, b):
            def f(u, c3):
                v = dbuf[pl.ds(jj * STREAM + u * 16, 16)]
                loc = v - base
                ok = (loc >= 0) & (loc < HALF)
                stg[b, pl.ds(u * 16, 16)] = jnp.where(ok, loc, TRASH)
                return c3

            lax.fori_loop(0, STREAM // 16, f, 0)

        def issue_gather(jj, b):
            pltpu.async_copy(y_hbm.at[sbuf.at[pl.ds(jj * STREAM, STREAM)]],
                             rows.at[b], gsem)

        def wait_gather(b):
            pltpu.make_async_copy(y_hbm.at[pl.ds(0, STREAM)], rows.at[b],
                                  gsem).wait()

        def issue_scatter(b):
            pltpu.async_copy(rows.at[b], acc.at[stg.at[b]], ssem, add=True)

        def drain_scatter():
            pltpu.make_async_copy(y_hbm.at[pl.ds(0, STREAM)], rows.at[0],
                                  ssem).wait()

        def mega(m, carry):
            ebase = ss * per_tile + m * MEGA_E
            pltpu.sync_copy(src_hbm.at[pl.ds(ebase, MEGA_E)], sbuf)
            pltpu.sync_copy(dst_hbm.at[pl.ds(ebase, MEGA_E)], dbuf)
            for b in range(NB - 1):
                fix(b, b)
                issue_gather(b, b)

            def grp(g, c2):
                for b in range(NB):
                    j = g * NB + b
                    wait_gather(b)

                    @pl.when(j > 0)
                    def _():
                        drain_scatter()

                    issue_scatter(b)
                    nb = (b + NB - 1) % NB

                    @pl.when(j + NB - 1 < NJ)
                    def _():
                        fix(j + NB - 1, nb)
                        issue_gather(j + NB - 1, nb)
                return c2

            lax.fori_loop(0, NG, grp, 0)
            drain_scatter()
            return carry

        lax.fori_loop(0, n_megas, mega, 0)
        plsc.subcore_barrier()
        pltpu.sync_copy(
            acc.at[pl.ds(ss * OUT_TPR, OUT_TPR)],
            out_hbm.at[pl.ds((cc * NS + ss) * OUT_TPR, OUT_TPR)],
        )

    return scat_kernel


BN = 2000


def _ln_rows(x, g, b):
    m = jnp.mean(x, axis=-1, keepdims=True)
    v = jnp.mean((x - m) ** 2, axis=-1, keepdims=True)
    return (x - m) * lax.rsqrt(v + LN_EPS) * g + b


def _dense_a_body(bw_ref, gu_ref, mu_ref, gf_ref, mc_ref, rid_ref, d0_ref, d1_ref,
                  remb_ref, wgf_ref, wmc_ref, bs_ref, wih_ref, bih_ref,
                  whh_ref, bhh_ref, lng_ref, lnb_ref, w1t_ref,
                  y1_ref, dinv_ref):
    gf = jnp.log1p(jnp.maximum(gf_ref[...], 0.0))
    mc = jnp.log1p(jnp.maximum(mc_ref[...], 0.0))
    rid = rid_ref[...]
    h_static = gf * wgf_ref[...] + mc * wmc_ref[...] + bs_ref[...]
    for k in range(3):
        h_static += jnp.where(rid == k, 1.0, 0.0) * remb_ref[k:k + 1, :]

    bw = jnp.log1p(jnp.maximum(bw_ref[...], 0.0))
    gu = gu_ref[...]
    mu = mu_ref[...]
    wih = wih_ref[...]
    bih = bih_ref[...]
    whh = whh_ref[...]
    bhh = bhh_ref[...]

    h = jnp.zeros((BN, D), jnp.float32)
    for t in range(T):
        gi = (bw[:, t:t + 1] * wih[0:1, :] + gu[:, t:t + 1] * wih[1:2, :]
              + mu[:, t:t + 1] * wih[2:3, :] + bih)
        gh = jnp.dot(h, whh, preferred_element_type=jnp.float32) + bhh
        r = jax.nn.sigmoid(gi[:, :D] + gh[:, :D])
        z = jax.nn.sigmoid(gi[:, D:2 * D] + gh[:, D:2 * D])
        n = jnp.tanh(gi[:, 2 * D:] + r * gh[:, 2 * D:])
        h = (1.0 - z) * n + z * h

    h_dyn = _ln_rows(h, lng_ref[...], lnb_ref[...])
    x0 = jnp.maximum(h_static + h_dyn, 0.0)
    dinv = lax.rsqrt(d0_ref[...] + d1_ref[...] + 1.0)
    y1_ref[...] = jnp.dot(x0, w1t_ref[...], preferred_element_type=jnp.float32) * dinv
    dinv_ref[...] = dinv


def _dense_a(bw, gu, mu, gf, mc, rid, d0, d1, remb, wgf, wmc, bs_row,
             wihT, bih_row, whhT, bhh_row, lng, lnb, w1t):
    nb = N // BN
    row = lambda i: (i, 0)
    full = lambda i: (0, 0)
    spec = lambda shape, im: pl.BlockSpec(shape, im)
    return pl.pallas_call(
        _dense_a_body,
        grid=(nb,),
        in_specs=[
            spec((BN, T), row), spec((BN, T), row), spec((BN, T), row),
            spec((BN, 1), row), spec((BN, 1), row), spec((BN, 1), row),
            spec((BN, 1), row), spec((BN, 1), row),
            spec((3, D), full), spec((1, D), full), spec((1, D), full),
            spec((1, D), full), spec((3, 3 * D), full), spec((1, 3 * D), full),
            spec((D, 3 * D), full), spec((1, 3 * D), full),
            spec((1, D), full), spec((1, D), full), spec((D, D), full),
        ],
        out_specs=[spec((BN, D), row), spec((BN, 1), row)],
        out_shape=[
            jax.ShapeDtypeStruct((N, D), jnp.float32),
            jax.ShapeDtypeStruct((N, 1), jnp.float32),
        ],
    )(bw, gu, mu, gf, mc, rid, d0, d1, remb, wgf, wmc, bs_row,
      wihT, bih_row, whhT, bhh_row, lng, lnb, w1t)


def _dense_b_body(acc_ref, y_ref, dinv_ref, b_ref, lng_ref, lnb_ref, w2t_ref, out_ref):
    dinv = dinv_ref[...]
    o = dinv * (acc_ref[...] + y_ref[...]) + b_ref[...]
    h1 = _ln_rows(jnp.maximum(o, 0.0), lng_ref[...], lnb_ref[...])
    out_ref[...] = jnp.dot(h1, w2t_ref[...], preferred_element_type=jnp.float32) * dinv


def _dense_b(acc1, y1, dinv, b_row, lng, lnb, w2t):
    nb = N // BN
    row = lambda i: (i, 0)
    full = lambda i: (0, 0)
    spec = lambda shape, im: pl.BlockSpec(shape, im)
    return pl.pallas_call(
        _dense_b_body,
        grid=(nb,),
        in_specs=[
            spec((BN, D), row), spec((BN, D), row), spec((BN, 1), row),
            spec((1, D), full), spec((1, D), full), spec((1, D), full),
            spec((D, D), full),
        ],
        out_specs=spec((BN, D), row),
        out_shape=jax.ShapeDtypeStruct((N, D), jnp.float32),
    )(acc1, y1, dinv, b_row, lng, lnb, w2t)


def _dense_c_body(acc_ref, y_ref, dinv_ref, b_ref, lng_ref, lnb_ref, out_ref):
    o = dinv_ref[...] * (acc_ref[...] + y_ref[...]) + b_ref[...]
    out_ref[...] = _ln_rows(o, lng_ref[...], lnb_ref[...])


def _dense_c(acc2, y2, dinv, b_row, lng, lnb):
    nb = N // BN
    row = lambda i: (i, 0)
    full = lambda i: (0, 0)
    spec = lambda shape, im: pl.BlockSpec(shape, im)
    return pl.pallas_call(
        _dense_c_body,
        grid=(nb,),
        in_specs=[
            spec((BN, D), row), spec((BN, D), row), spec((BN, 1), row),
            spec((1, D), full), spec((1, D), full), spec((1, D), full),
        ],
        out_specs=spec((BN, D), row),
        out_shape=jax.ShapeDtypeStruct((N, D), jnp.float32),
    )(acc2, y2, dinv, b_row, lng, lnb)


def kernel(edge_index, gpu_flops, role_id, mem_capacity, bandwidth_seq,
           gpu_util_seq, mem_util_seq, role_emb, Ws, bs, Wih, Whh, bih, bhh,
           lnt_g, lnt_b, W1, b1, W2, b2, ln1_g, ln1_b, ln2_g, ln2_b):
    E = edge_index.shape[1]
    grain = NS * MEGA_E
    e_pad = ((E + grain - 1) // grain) * grain

    src = edge_index[0]
    dst = edge_index[1]
    src_p = jnp.concatenate([src, jnp.zeros((e_pad - E,), jnp.int32)])
    dst_p = jnp.concatenate([dst, jnp.full((e_pad - E,), N, jnp.int32)])

    ones_col = jnp.ones((128, 1), jnp.float32)
    zeros_deg = jnp.zeros((DEG_TPR, 1), jnp.float32)
    zeros_acc = jnp.zeros((SP_TPR, D), jnp.float32)

    deg_kernel = _make_degree_kernel(e_pad)
    deg_parts = deg_kernel(dst_p, ones_col, zeros_deg)
    d0 = deg_parts[:N]
    d1 = deg_parts[DEG_ROWS:DEG_ROWS + N]

    WsT = Ws.T
    wgf = WsT[0:1, :]
    wmc = WsT[1:2, :]
    remb = role_emb @ WsT[2:10, :]
    bs_row = bs[None, :]
    wihT = Wih.T
    whhT = Whh.T
    bih_row = bih[None, :]
    bhh_row = bhh[None, :]

    y1, dinv = _dense_a(
        bandwidth_seq, gpu_util_seq, mem_util_seq,
        gpu_flops[:, None], mem_capacity[:, None], role_id[:, None],
        d0, d1, remb, wgf, wmc, bs_row, wihT, bih_row, whhT, bhh_row,
        lnt_g[None, :], lnt_b[None, :], W1.T)

    scat_kernel = _make_scatter_kernel(e_pad)

    acc1_raw = scat_kernel(src_p, dst_p, y1, zeros_acc)
    acc1 = jnp.concatenate(
        [acc1_raw[:HALF], acc1_raw[NS * OUT_TPR:NS * OUT_TPR + HALF]], axis=0)

    y2 = _dense_b(acc1, y1, dinv, b1[None, :], ln1_g[None, :], ln1_b[None, :], W2.T)

    acc2_raw = scat_kernel(src_p, dst_p, y2, zeros_acc)
    acc2 = jnp.concatenate(
        [acc2_raw[:HALF], acc2_raw[NS * OUT_TPR:NS * OUT_TPR + HALF]], axis=0)

    return _dense_c(acc2, y2, dinv, b2[None, :], ln2_g[None, :], ln2_b[None, :])


# speedup vs baseline: 7.4335x; 1.1082x over previous
import functools

import jax
import jax.numpy as jnp
from jax import lax
from jax.experimental import pallas as pl
from jax.experimental.pallas import tpu as pltpu
from jax.experimental.pallas import tpu_sc as plsc

N = 50000
D = 64
T = 20
NC = 2
NS = 16
LN_EPS = 1e-5

HALF = 25000
OUT_TPR = 1564
SP_TPR = 1564
SP_ROWS = NS * SP_TPR
TRASH = 25000

DEG_TPR = 3128
DEG_ROWS = NS * DEG_TPR

STREAM = 64
MEGA_E = 2560
NJ = MEGA_E // STREAM
NB = 5
NG = NJ // NB


def _sc_mesh():
    return plsc.VectorSubcoreMesh(
        core_axis_name="c", subcore_axis_name="s", num_cores=NC, num_subcores=NS
    )


_SC_PARAMS = pltpu.CompilerParams(use_tc_tiling_on_sc=False)


def _make_degree_kernel(e_pad):
    per_tile = e_pad // (NC * NS)
    n_j = per_tile // 128

    @functools.partial(
        pl.kernel,
        out_type=jax.ShapeDtypeStruct((NC * DEG_ROWS, 1), jnp.float32),
        mesh=_sc_mesh(),
        scratch_types=[
            pltpu.VMEM((per_tile,), jnp.int32),
            pltpu.VMEM((1, 128), jnp.int32),
            pltpu.VMEM((128, 1), jnp.float32),
            pltpu.VMEM((DEG_TPR, 1), jnp.float32),
            pltpu.VMEM_SHARED((DEG_ROWS, 1), jnp.float32),
        ],
        compiler_params=_SC_PARAMS,
    )
    def deg_kernel(dst_hbm, ones_hbm, zeros_hbm, out_hbm, dbuf, stg, ones_v, z_v, hist):
        cc = lax.axis_index("c")
        ss = lax.axis_index("s")
        pltpu.sync_copy(ones_hbm, ones_v)
        pltpu.sync_copy(zeros_hbm, z_v)
        pltpu.sync_copy(z_v, hist.at[pl.ds(ss * DEG_TPR, DEG_TPR)])
        plsc.subcore_barrier()

        ebase = (cc * NS + ss) * per_tile
        pltpu.sync_copy(dst_hbm.at[pl.ds(ebase, per_tile)], dbuf)

        def body(j, carry):
            def fix(u, c2):
                stg[0, pl.ds(u * 16, 16)] = dbuf[pl.ds(j * 128 + u * 16, 16)]
                return c2

            lax.fori_loop(0, 8, fix, 0)
            pltpu.sync_copy(ones_v, hist.at[stg.at[0]], add=True)
            return carry

        lax.fori_loop(0, n_j, body, 0)
        plsc.subcore_barrier()
        pltpu.sync_copy(
            hist.at[pl.ds(ss * DEG_TPR, DEG_TPR)],
            out_hbm.at[pl.ds(cc * DEG_ROWS + ss * DEG_TPR, DEG_TPR)],
        )

    return deg_kernel


def _make_scatter_kernel(e_pad):
    per_tile = e_pad // NS
    n_megas = per_tile // MEGA_E

    @functools.partial(
        pl.kernel,
        out_type=jax.ShapeDtypeStruct((NC * NS * OUT_TPR, D), jnp.float32),
        mesh=_sc_mesh(),
        scratch_types=[
            pltpu.VMEM((MEGA_E,), jnp.int32),
            pltpu.VMEM((MEGA_E,), jnp.int32),
            pltpu.VMEM((NB, STREAM), jnp.int32),
            pltpu.VMEM((NB, STREAM, D), jnp.float32),
            pltpu.SemaphoreType.DMA,
            pltpu.SemaphoreType.DMA,
            pltpu.VMEM_SHARED((SP_ROWS, D), jnp.float32),
        ],
        compiler_params=_SC_PARAMS,
    )
    def scat_kernel(src_hbm, dst_hbm, y_hbm, zeros_hbm, out_hbm,
                    sbuf, dbuf, stg, rows, gsem, ssem, acc):
        cc = lax.axis_index("c")
        ss = lax.axis_index("s")
        pltpu.sync_copy(zeros_hbm, acc.at[pl.ds(ss * SP_TPR, SP_TPR)])
        plsc.subcore_barrier()

        base = cc * HALF

        def fix(jj, b):
            def f(u, c3):
                v = dbuf[pl.ds(jj * STREAM + u * 16, 16)]
                loc = v - base
                ok = (loc >= 0) & (loc < HALF)
                stg[b, pl.ds(u * 16, 16)] = jnp.where(ok, loc, TRASH)
                return c3

            lax.fori_loop(0, STREAM // 16, f, 0)

        def issue_gather(jj, b):
            pltpu.async_copy(y_hbm.at[sbuf.at[pl.ds(jj * STREAM, STREAM)]],
                             rows.at[b], gsem)

        def wait_gather(b):
            pltpu.make_async_copy(y_hbm.at[pl.ds(0, STREAM)], rows.at[b],
                                  gsem).wait()

        def issue_scatter(b):
            pltpu.async_copy(rows.at[b], acc.at[stg.at[b]], ssem, add=True)

        def drain_scatter():
            pltpu.make_async_copy(y_hbm.at[pl.ds(0, STREAM)], rows.at[0],
                                  ssem).wait()

        def mega(m, carry):
            ebase = ss * per_tile + m * MEGA_E
            pltpu.sync_copy(src_hbm.at[pl.ds(ebase, MEGA_E)], sbuf)
            pltpu.sync_copy(dst_hbm.at[pl.ds(ebase, MEGA_E)], dbuf)
            for b in range(NB - 1):
                fix(b, b)
                issue_gather(b, b)

            def grp(g, c2):
                for b in range(NB):
                    j = g * NB + b
                    wait_gather(b)

                    @pl.when(j > 0)
                    def _():
                        drain_scatter()

                    issue_scatter(b)
                    nb = (b + NB - 1) % NB

                    @pl.when(j + NB - 1 < NJ)
                    def _():
                        fix(j + NB - 1, nb)
                        issue_gather(j + NB - 1, nb)
                return c2

            lax.fori_loop(0, NG, grp, 0)
            drain_scatter()
            return carry

        lax.fori_loop(0, n_megas, mega, 0)
        plsc.subcore_barrier()
        pltpu.sync_copy(
            acc.at[pl.ds(ss * OUT_TPR, OUT_TPR)],
            out_hbm.at[pl.ds((cc * NS + ss) * OUT_TPR, OUT_TPR)],
        )

    return scat_kernel


BN = 2000


def _ln_rows(x, g, b):
    m = jnp.mean(x, axis=-1, keepdims=True)
    v = jnp.mean((x - m) ** 2, axis=-1, keepdims=True)
    return (x - m) * lax.rsqrt(v + LN_EPS) * g + b


def _dense_a_body(bw_ref, gu_ref, mu_ref, gf_ref, mc_ref, rid_ref, d0_ref, d1_ref,
                  remb_ref, wgf_ref, wmc_ref, bs_ref, wcat_ref, bcat_ref,
                  lng_ref, lnb_ref, w1t_ref,
                  y1_ref, dinv_ref):
    gf = jnp.log1p(jnp.maximum(gf_ref[...], 0.0))
    mc = jnp.log1p(jnp.maximum(mc_ref[...], 0.0))
    rid = rid_ref[...]
    h_static = gf * wgf_ref[...] + mc * wmc_ref[...] + bs_ref[...]
    for k in range(3):
        h_static += jnp.where(rid == k, 1.0, 0.0) * remb_ref[k:k + 1, :]

    bw = jnp.log1p(jnp.maximum(bw_ref[...], 0.0))
    gu = gu_ref[...]
    mu = mu_ref[...]
    wcat = wcat_ref[...]
    bcat = bcat_ref[...]

    h = jnp.zeros((BN, D), jnp.float32)
    for t in range(T):
        x_cat = jnp.concatenate(
            [bw[:, t:t + 1], gu[:, t:t + 1], mu[:, t:t + 1], h], axis=-1)
        g = jnp.dot(x_cat, wcat, preferred_element_type=jnp.float32) + bcat
        rz = jax.nn.sigmoid(g[:, :2 * D])
        r = rz[:, :D]
        z = rz[:, D:2 * D]
        n = jnp.tanh(g[:, 3 * D:] + r * g[:, 2 * D:3 * D])
        h = n + z * (h - n)

    h_dyn = _ln_rows(h, lng_ref[...], lnb_ref[...])
    x0 = jnp.maximum(h_static + h_dyn, 0.0)
    dinv = lax.rsqrt(d0_ref[...] + d1_ref[...] + 1.0)
    y1_ref[...] = jnp.dot(x0, w1t_ref[...], preferred_element_type=jnp.float32) * dinv
    dinv_ref[...] = dinv


def _dense_a(bw, gu, mu, gf, mc, rid, d0, d1, remb, wgf, wmc, bs_row,
             wcat, bcat, lng, lnb, w1t):
    nb = N // BN
    row = lambda i: (i, 0)
    full = lambda i: (0, 0)
    spec = lambda shape, im: pl.BlockSpec(shape, im)
    return pl.pallas_call(
        _dense_a_body,
        grid=(nb,),
        in_specs=[
            spec((BN, T), row), spec((BN, T), row), spec((BN, T), row),
            spec((BN, 1), row), spec((BN, 1), row), spec((BN, 1), row),
            spec((BN, 1), row), spec((BN, 1), row),
            spec((3, D), full), spec((1, D), full), spec((1, D), full),
            spec((1, D), full), spec((3 + D, 4 * D), full), spec((1, 4 * D), full),
            spec((1, D), full), spec((1, D), full), spec((D, D), full),
        ],
        out_specs=[spec((BN, D), row), spec((BN, 1), row)],
        out_shape=[
            jax.ShapeDtypeStruct((N, D), jnp.float32),
            jax.ShapeDtypeStruct((N, 1), jnp.float32),
        ],
    )(bw, gu, mu, gf, mc, rid, d0, d1, remb, wgf, wmc, bs_row,
      wcat, bcat, lng, lnb, w1t)


def _dense_b_body(acc_ref, y_ref, dinv_ref, b_ref, lng_ref, lnb_ref, w2t_ref, out_ref):
    dinv = dinv_ref[...]
    o = dinv * (acc_ref[...] + y_ref[...]) + b_ref[...]
    h1 = _ln_rows(jnp.maximum(o, 0.0), lng_ref[...], lnb_ref[...])
    out_ref[...] = jnp.dot(h1, w2t_ref[...], preferred_element_type=jnp.float32) * dinv


def _dense_b(acc1, y1, dinv, b_row, lng, lnb, w2t):
    nb = N // BN
    row = lambda i: (i, 0)
    full = lambda i: (0, 0)
    spec = lambda shape, im: pl.BlockSpec(shape, im)
    return pl.pallas_call(
        _dense_b_body,
        grid=(nb,),
        in_specs=[
            spec((BN, D), row), spec((BN, D), row), spec((BN, 1), row),
            spec((1, D), full), spec((1, D), full), spec((1, D), full),
            spec((D, D), full),
        ],
        out_specs=spec((BN, D), row),
        out_shape=jax.ShapeDtypeStruct((N, D), jnp.float32),
    )(acc1, y1, dinv, b_row, lng, lnb, w2t)


def _dense_c_body(acc_ref, y_ref, dinv_ref, b_ref, lng_ref, lnb_ref, out_ref):
    o = dinv_ref[...] * (acc_ref[...] + y_ref[...]) + b_ref[...]
    out_ref[...] = _ln_rows(o, lng_ref[...], lnb_ref[...])


def _dense_c(acc2, y2, dinv, b_row, lng, lnb):
    nb = N // BN
    row = lambda i: (i, 0)
    full = lambda i: (0, 0)
    spec = lambda shape, im: pl.BlockSpec(shape, im)
    return pl.pallas_call(
        _dense_c_body,
        grid=(nb,),
        in_specs=[
            spec((BN, D), row), spec((BN, D), row), spec((BN, 1), row),
            spec((1, D), full), spec((1, D), full), spec((1, D), full),
        ],
        out_specs=spec((BN, D), row),
        out_shape=jax.ShapeDtypeStruct((N, D), jnp.float32),
    )(acc2, y2, dinv, b_row, lng, lnb)


def kernel(edge_index, gpu_flops, role_id, mem_capacity, bandwidth_seq,
           gpu_util_seq, mem_util_seq, role_emb, Ws, bs, Wih, Whh, bih, bhh,
           lnt_g, lnt_b, W1, b1, W2, b2, ln1_g, ln1_b, ln2_g, ln2_b):
    E = edge_index.shape[1]
    grain = NS * MEGA_E
    e_pad = ((E + grain - 1) // grain) * grain

    src = edge_index[0]
    dst = edge_index[1]
    src_p = jnp.concatenate([src, jnp.zeros((e_pad - E,), jnp.int32)])
    dst_p = jnp.concatenate([dst, jnp.full((e_pad - E,), N, jnp.int32)])

    ones_col = jnp.ones((128, 1), jnp.float32)
    zeros_deg = jnp.zeros((DEG_TPR, 1), jnp.float32)
    zeros_acc = jnp.zeros((SP_TPR, D), jnp.float32)

    deg_kernel = _make_degree_kernel(e_pad)
    deg_parts = deg_kernel(dst_p, ones_col, zeros_deg)
    d0 = deg_parts[:N]
    d1 = deg_parts[DEG_ROWS:DEG_ROWS + N]

    WsT = Ws.T
    wgf = WsT[0:1, :]
    wmc = WsT[1:2, :]
    remb = role_emb @ WsT[2:10, :]
    bs_row = bs[None, :]
    wihT = Wih.T
    whhT = Whh.T
    z3 = jnp.zeros((3, D), jnp.float32)
    zD = jnp.zeros((D, D), jnp.float32)
    wcat = jnp.concatenate([
        jnp.concatenate([wihT[:, :D], wihT[:, D:2 * D], z3, wihT[:, 2 * D:]], 1),
        jnp.concatenate([whhT[:, :D], whhT[:, D:2 * D], whhT[:, 2 * D:], zD], 1),
    ], axis=0)
    bcat = jnp.concatenate(
        [bih[:D] + bhh[:D], bih[D:2 * D] + bhh[D:2 * D],
         bhh[2 * D:], bih[2 * D:]])[None, :]

    y1, dinv = _dense_a(
        bandwidth_seq, gpu_util_seq, mem_util_seq,
        gpu_flops[:, None], mem_capacity[:, None], role_id[:, None],
        d0, d1, remb, wgf, wmc, bs_row, wcat, bcat,
        lnt_g[None, :], lnt_b[None, :], W1.T)

    scat_kernel = _make_scatter_kernel(e_pad)

    acc1_raw = scat_kernel(src_p, dst_p, y1, zeros_acc)
    acc1 = jnp.concatenate(
        [acc1_raw[:HALF], acc1_raw[NS * OUT_TPR:NS * OUT_TPR + HALF]], axis=0)

    y2 = _dense_b(acc1, y1, dinv, b1[None, :], ln1_g[None, :], ln1_b[None, :], W2.T)

    acc2_raw = scat_kernel(src_p, dst_p, y2, zeros_acc)
    acc2 = jnp.concatenate(
        [acc2_raw[:HALF], acc2_raw[NS * OUT_TPR:NS * OUT_TPR + HALF]], axis=0)

    return _dense_c(acc2, y2, dinv, b2[None, :], ln2_g[None, :], ln2_b[None, :])


# speedup vs baseline: 8.3496x; 1.1232x over previous
import functools

import jax
import jax.numpy as jnp
from jax import lax
from jax.experimental import pallas as pl
from jax.experimental.pallas import tpu as pltpu
from jax.experimental.pallas import tpu_sc as plsc

N = 50000
D = 64
T = 20
NC = 2
NS = 16
LN_EPS = 1e-5

HALF = 25000
OUT_TPR = 1564
SP_TPR = 1564
SP_ROWS = NS * SP_TPR
TRASH = 25000

DEG_TPR = 3128
DEG_ROWS = NS * DEG_TPR

STREAM = 64
MEGA_E = 2560
NJ = MEGA_E // STREAM
NB = 5
NG = NJ // NB

CHUNK = 2048
RB_CAP = 6672
CAP_ROW = 59392


def _sc_mesh():
    return plsc.VectorSubcoreMesh(
        core_axis_name="c", subcore_axis_name="s", num_cores=NC, num_subcores=NS
    )


_SC_PARAMS = pltpu.CompilerParams(use_tc_tiling_on_sc=False)
_SC_PARAMS_NL = pltpu.CompilerParams(use_tc_tiling_on_sc=False,
                                     needs_layout_passes=False)


def _make_degree_kernel(e_pad):
    per_tile = e_pad // (NC * NS)
    n_j = per_tile // 128

    @functools.partial(
        pl.kernel,
        out_type=jax.ShapeDtypeStruct((NC * DEG_ROWS, 1), jnp.float32),
        mesh=_sc_mesh(),
        scratch_types=[
            pltpu.VMEM((per_tile,), jnp.int32),
            pltpu.VMEM((1, 128), jnp.int32),
            pltpu.VMEM((128, 1), jnp.float32),
            pltpu.VMEM((DEG_TPR, 1), jnp.float32),
            pltpu.VMEM_SHARED((DEG_ROWS, 1), jnp.float32),
        ],
        compiler_params=_SC_PARAMS,
    )
    def deg_kernel(dst_hbm, ones_hbm, zeros_hbm, out_hbm, dbuf, stg, ones_v, z_v, hist):
        cc = lax.axis_index("c")
        ss = lax.axis_index("s")
        pltpu.sync_copy(ones_hbm, ones_v)
        pltpu.sync_copy(zeros_hbm, z_v)
        pltpu.sync_copy(z_v, hist.at[pl.ds(ss * DEG_TPR, DEG_TPR)])
        plsc.subcore_barrier()

        ebase = (cc * NS + ss) * per_tile
        pltpu.sync_copy(dst_hbm.at[pl.ds(ebase, per_tile)], dbuf)

        def body(j, carry):
            def fix(u, c2):
                stg[0, pl.ds(u * 16, 16)] = dbuf[pl.ds(j * 128 + u * 16, 16)]
                return c2

            lax.fori_loop(0, 8, fix, 0)
            pltpu.sync_copy(ones_v, hist.at[stg.at[0]], add=True)
            return carry

        lax.fori_loop(0, n_j, body, 0)
        plsc.subcore_barrier()
        pltpu.sync_copy(
            hist.at[pl.ds(ss * DEG_TPR, DEG_TPR)],
            out_hbm.at[pl.ds(cc * DEG_ROWS + ss * DEG_TPR, DEG_TPR)],
        )

    return deg_kernel


def _make_prep_kernel(e_pad):
    per_tile = e_pad // NS
    n_chunks = per_tile // CHUNK

    @functools.partial(
        pl.kernel,
        out_type=[
            jax.ShapeDtypeStruct((NC * NS * CAP_ROW,), jnp.int32),
            jax.ShapeDtypeStruct((NC * NS * CAP_ROW,), jnp.int32),
            jax.ShapeDtypeStruct((NC * NS, 16), jnp.int32),
        ],
        mesh=_sc_mesh(),
        scratch_types=[
            pltpu.VMEM((CHUNK,), jnp.int32),
            pltpu.VMEM((CHUNK,), jnp.int32),
            pltpu.VMEM((RB_CAP,), jnp.int32),
            pltpu.VMEM((RB_CAP,), jnp.int32),
            pltpu.VMEM((1, 16), jnp.int32),
        ],
        compiler_params=_SC_PARAMS_NL,
    )
    def prep_kernel(src_hbm, dst_hbm, es_hbm, ed_hbm, cnt_hbm,
                    in_s, in_d, rb_s, rb_d, cbuf):
        cc = lax.axis_index("c")
        ss = lax.axis_index("s")
        base = cc * HALF
        out0 = (cc * NS + ss) * CAP_ROW

        def flush_block(off):
            off = pl.multiple_of(off, CHUNK)
            pltpu.sync_copy(rb_s.at[pl.ds(0, CHUNK)],
                            es_hbm.at[pl.ds(out0 + off, CHUNK)])
            pltpu.sync_copy(rb_d.at[pl.ds(0, CHUNK)],
                            ed_hbm.at[pl.ds(out0 + off, CHUNK)])

            def mv(i, c4):
                rb_s[pl.ds(i * 16, 16)] = rb_s[pl.ds(CHUNK + i * 16, 16)]
                rb_d[pl.ds(i * 16, 16)] = rb_d[pl.ds(CHUNK + i * 16, 16)]
                return c4

            lax.fori_loop(0, (RB_CAP - CHUNK) // 16, mv, 0)

        lane15 = jnp.full((16,), 15, jnp.int32)
        iota16 = lax.iota(jnp.int32, 16)

        def chunk(ci, carry):
            pend_v, off = carry
            ebase = ss * per_tile + ci * CHUNK
            pltpu.sync_copy(src_hbm.at[pl.ds(ebase, CHUNK)], in_s)
            pltpu.sync_copy(dst_hbm.at[pl.ds(ebase, CHUNK)], in_d)

            def vec(v, pend2):
                sv = in_s[pl.ds(v * 16, 16)]
                dv = in_d[pl.ds(v * 16, 16)]
                loc = dv - base
                ok = (loc >= 0) & (loc < HALF)
                okv = jnp.where(ok, 1, 0).astype(jnp.int32)
                incl = plsc.cumsum(okv)
                pos = pend2 + incl - okv
                plsc.store_scatter(rb_s, [pos], sv, mask=ok)
                plsc.store_scatter(rb_d, [pos], loc, mask=ok)
                return pend2 + incl[lane15]

            pend_v = lax.fori_loop(0, CHUNK // 16, vec, pend_v)
            pend_s = lax.reduce_max(pend_v, (0,))

            def do_flush(args):
                pv3, o3 = args
                flush_block(o3)
                return pv3 - CHUNK, o3 + CHUNK

            pend_v, off = lax.cond(pend_s >= CHUNK, do_flush, lambda a: a,
                                   (pend_v, off))
            return pend_v, off

        pend_v, off = lax.fori_loop(
            0, n_chunks, chunk, (jnp.zeros((16,), jnp.int32), 0))
        pend = lax.reduce_max(pend_v, (0,))
        cnt = off + pend

        def padv(i, c2):
            idxv = pend_v + i * 16 + iota16
            plsc.store_scatter(rb_s, [idxv], jnp.zeros((16,), jnp.int32))
            plsc.store_scatter(rb_d, [idxv], jnp.full((16,), TRASH, jnp.int32))
            return c2

        lax.fori_loop(0, MEGA_E // 16, padv, 0)

        def fin(k, off5):
            flush_block(off5)
            return off5 + CHUNK

        lax.fori_loop(0, 3, fin, off)

        cbuf[0, pl.ds(0, 16)] = jnp.full((16,), 1, jnp.int32) * cnt
        pltpu.sync_copy(cbuf, cnt_hbm.at[pl.ds(cc * NS + ss, 1)])

    return prep_kernel


def _make_scatter_kernel(e_pad):
    per_tile = e_pad // NS
    n_megas = per_tile // MEGA_E

    @functools.partial(
        pl.kernel,
        out_type=jax.ShapeDtypeStruct((NC * NS * OUT_TPR, D), jnp.float32),
        mesh=_sc_mesh(),
        scratch_types=[
            pltpu.VMEM((MEGA_E,), jnp.int32),
            pltpu.VMEM((MEGA_E,), jnp.int32),
            pltpu.VMEM((NB, STREAM), jnp.int32),
            pltpu.VMEM((NB, STREAM, D), jnp.float32),
            pltpu.VMEM((1, 16), jnp.int32),
            pltpu.SemaphoreType.DMA,
            pltpu.SemaphoreType.DMA,
            pltpu.VMEM_SHARED((SP_ROWS, D), jnp.float32),
        ],
        compiler_params=_SC_PARAMS_NL,
    )
    def scat_kernel(es_hbm, ed_hbm, cnt_hbm, y_hbm, zeros_hbm, out_hbm,
                    sbuf, dbuf, stg, rows, cbuf, gsem, ssem, acc):
        cc = lax.axis_index("c")
        ss = lax.axis_index("s")
        pltpu.sync_copy(zeros_hbm, acc.at[pl.ds(ss * SP_TPR, SP_TPR)])
        pltpu.sync_copy(cnt_hbm.at[pl.ds(cc * NS + ss, 1)], cbuf)
        plsc.subcore_barrier()

        out0 = (cc * NS + ss) * CAP_ROW
        cnt = lax.reduce_max(cbuf[0, pl.ds(0, 16)], (0,))
        nm = (cnt + MEGA_E - 1) // MEGA_E

        def fix(jj, b):
            def f(u, c3):
                stg[b, pl.ds(u * 16, 16)] = dbuf[pl.ds(jj * STREAM + u * 16, 16)]
                return c3

            lax.fori_loop(0, STREAM // 16, f, 0)

        def issue_gather(jj, b):
            pltpu.async_copy(y_hbm.at[sbuf.at[pl.ds(jj * STREAM, STREAM)]],
                             rows.at[b], gsem)

        def wait_gather(b):
            pltpu.make_async_copy(y_hbm.at[pl.ds(0, STREAM)], rows.at[b],
                                  gsem).wait()

        def issue_scatter(b):
            pltpu.async_copy(rows.at[b], acc.at[stg.at[b]], ssem, add=True)

        def drain_scatter():
            pltpu.make_async_copy(y_hbm.at[pl.ds(0, STREAM)], rows.at[0],
                                  ssem).wait()

        def mega(m, carry):
            ebase = out0 + m * MEGA_E
            pltpu.sync_copy(es_hbm.at[pl.ds(ebase, MEGA_E)], sbuf)
            pltpu.sync_copy(ed_hbm.at[pl.ds(ebase, MEGA_E)], dbuf)
            for b in range(NB - 1):
                fix(b, b)
                issue_gather(b, b)

            def grp(g, c2):
                for b in range(NB):
                    j = g * NB + b
                    wait_gather(b)

                    @pl.when(j > 0)
                    def _():
                        drain_scatter()

                    issue_scatter(b)
                    nb = (b + NB - 1) % NB

                    @pl.when(j + NB - 1 < NJ)
                    def _():
                        fix(j + NB - 1, nb)
                        issue_gather(j + NB - 1, nb)
                return c2

            lax.fori_loop(0, NG, grp, 0)
            drain_scatter()
            return carry

        lax.fori_loop(0, nm, mega, 0)
        plsc.subcore_barrier()
        pltpu.sync_copy(
            acc.at[pl.ds(ss * OUT_TPR, OUT_TPR)],
            out_hbm.at[pl.ds((cc * NS + ss) * OUT_TPR, OUT_TPR)],
        )

    return scat_kernel


BN = 2000


def _ln_rows(x, g, b):
    m = jnp.mean(x, axis=-1, keepdims=True)
    v = jnp.mean((x - m) ** 2, axis=-1, keepdims=True)
    return (x - m) / jnp.sqrt(v + LN_EPS) * g + b


def _dense_a_body(bw_ref, gu_ref, mu_ref, gf_ref, mc_ref, rid_ref, d0_ref, d1_ref,
                  remb_ref, wgf_ref, wmc_ref, bs_ref, wcat_ref, bcat_ref,
                  lng_ref, lnb_ref, w1t_ref,
                  y1_ref, dinv_ref):
    gf = jnp.log1p(jnp.maximum(gf_ref[...], 0.0))
    mc = jnp.log1p(jnp.maximum(mc_ref[...], 0.0))
    rid = rid_ref[...]
    h_static = gf * wgf_ref[...] + mc * wmc_ref[...] + bs_ref[...]
    for k in range(3):
        h_static += jnp.where(rid == k, 1.0, 0.0) * remb_ref[k:k + 1, :]

    bw = jnp.log1p(jnp.maximum(bw_ref[...], 0.0))
    gu = gu_ref[...]
    mu = mu_ref[...]
    wcat = wcat_ref[...]
    bcat = bcat_ref[...]

    h = jnp.zeros((BN, D), jnp.float32)
    for t in range(T):
        x_cat = jnp.concatenate(
            [bw[:, t:t + 1], gu[:, t:t + 1], mu[:, t:t + 1], h], axis=-1)
        g = jnp.dot(x_cat, wcat, preferred_element_type=jnp.float32) + bcat
        rz = jax.nn.sigmoid(g[:, :2 * D])
        r = rz[:, :D]
        z = rz[:, D:2 * D]
        n = jnp.tanh(g[:, 3 * D:] + r * g[:, 2 * D:3 * D])
        h = n + z * (h - n)

    h_dyn = _ln_rows(h, lng_ref[...], lnb_ref[...])
    x0 = jnp.maximum(h_static + h_dyn, 0.0)
    dinv = 1.0 / jnp.sqrt(d0_ref[...] + d1_ref[...] + 1.0)
    y1_ref[...] = jnp.dot(x0, w1t_ref[...], preferred_element_type=jnp.float32) * dinv
    dinv_ref[...] = dinv


def _dense_a(bw, gu, mu, gf, mc, rid, d0, d1, remb, wgf, wmc, bs_row,
             wcat, bcat, lng, lnb, w1t):
    nb = N // BN
    row = lambda i: (i, 0)
    full = lambda i: (0, 0)
    spec = lambda shape, im: pl.BlockSpec(shape, im)
    return pl.pallas_call(
        _dense_a_body,
        grid=(nb,),
        in_specs=[
            spec((BN, T), row), spec((BN, T), row), spec((BN, T), row),
            spec((BN, 1), row), spec((BN, 1), row), spec((BN, 1), row),
            spec((BN, 1), row), spec((BN, 1), row),
            spec((3, D), full), spec((1, D), full), spec((1, D), full),
            spec((1, D), full), spec((3 + D, 4 * D), full), spec((1, 4 * D), full),
            spec((1, D), full), spec((1, D), full), spec((D, D), full),
        ],
        out_specs=[spec((BN, D), row), spec((BN, 1), row)],
        out_shape=[
            jax.ShapeDtypeStruct((N, D), jnp.float32),
            jax.ShapeDtypeStruct((N, 1), jnp.float32),
        ],
    )(bw, gu, mu, gf, mc, rid, d0, d1, remb, wgf, wmc, bs_row,
      wcat, bcat, lng, lnb, w1t)


def _dense_b_body(acc_ref, y_ref, dinv_ref, b_ref, lng_ref, lnb_ref, w2t_ref, out_ref):
    dinv = dinv_ref[...]
    o = dinv * (acc_ref[...] + y_ref[...]) + b_ref[...]
    h1 = _ln_rows(jnp.maximum(o, 0.0), lng_ref[...], lnb_ref[...])
    out_ref[...] = jnp.dot(h1, w2t_ref[...], preferred_element_type=jnp.float32) * dinv


def _dense_b(acc1, y1, dinv, b_row, lng, lnb, w2t):
    nb = N // BN
    row = lambda i: (i, 0)
    full = lambda i: (0, 0)
    spec = lambda shape, im: pl.BlockSpec(shape, im)
    return pl.pallas_call(
        _dense_b_body,
        grid=(nb,),
        in_specs=[
            spec((BN, D), row), spec((BN, D), row), spec((BN, 1), row),
            spec((1, D), full), spec((1, D), full), spec((1, D), full),
            spec((D, D), full),
        ],
        out_specs=spec((BN, D), row),
        out_shape=jax.ShapeDtypeStruct((N, D), jnp.float32),
    )(acc1, y1, dinv, b_row, lng, lnb, w2t)


def _dense_c_body(acc_ref, y_ref, dinv_ref, b_ref, lng_ref, lnb_ref, out_ref):
    o = dinv_ref[...] * (acc_ref[...] + y_ref[...]) + b_ref[...]
    out_ref[...] = _ln_rows(o, lng_ref[...], lnb_ref[...])


def _dense_c(acc2, y2, dinv, b_row, lng, lnb):
    nb = N // BN
    row = lambda i: (i, 0)
    full = lambda i: (0, 0)
    spec = lambda shape, im: pl.BlockSpec(shape, im)
    return pl.pallas_call(
        _dense_c_body,
        grid=(nb,),
        in_specs=[
            spec((BN, D), row), spec((BN, D), row), spec((BN, 1), row),
            spec((1, D), full), spec((1, D), full), spec((1, D), full),
        ],
        out_specs=spec((BN, D), row),
        out_shape=jax.ShapeDtypeStruct((N, D), jnp.float32),
    )(acc2, y2, dinv, b_row, lng, lnb)


def kernel(edge_index, gpu_flops, role_id, mem_capacity, bandwidth_seq,
           gpu_util_seq, mem_util_seq, role_emb, Ws, bs, Wih, Whh, bih, bhh,
           lnt_g, lnt_b, W1, b1, W2, b2, ln1_g, ln1_b, ln2_g, ln2_b):
    E = edge_index.shape[1]
    grain = NS * MEGA_E
    e_pad = ((E + grain - 1) // grain) * grain

    src = edge_index[0]
    dst = edge_index[1]
    src_p = jnp.concatenate([src, jnp.zeros((e_pad - E,), jnp.int32)])
    dst_p = jnp.concatenate([dst, jnp.full((e_pad - E,), N, jnp.int32)])

    ones_col = jnp.ones((128, 1), jnp.float32)
    zeros_deg = jnp.zeros((DEG_TPR, 1), jnp.float32)
    zeros_acc = jnp.zeros((SP_TPR, D), jnp.float32)

    deg_kernel = _make_degree_kernel(e_pad)
    deg_parts = deg_kernel(dst_p, ones_col, zeros_deg)
    d0 = deg_parts[:N]
    d1 = deg_parts[DEG_ROWS:DEG_ROWS + N]

    WsT = Ws.T
    wgf = WsT[0:1, :]
    wmc = WsT[1:2, :]
    remb = role_emb @ WsT[2:10, :]
    bs_row = bs[None, :]
    wihT = Wih.T
    whhT = Whh.T
    z3 = jnp.zeros((3, D), jnp.float32)
    zD = jnp.zeros((D, D), jnp.float32)
    wcat = jnp.concatenate([
        jnp.concatenate([wihT[:, :D], wihT[:, D:2 * D], z3, wihT[:, 2 * D:]], 1),
        jnp.concatenate([whhT[:, :D], whhT[:, D:2 * D], whhT[:, 2 * D:], zD], 1),
    ], axis=0)
    bcat = jnp.concatenate(
        [bih[:D] + bhh[:D], bih[D:2 * D] + bhh[D:2 * D],
         bhh[2 * D:], bih[2 * D:]])[None, :]

    y1, dinv = _dense_a(
        bandwidth_seq, gpu_util_seq, mem_util_seq,
        gpu_flops[:, None], mem_capacity[:, None], role_id[:, None],
        d0, d1, remb, wgf, wmc, bs_row, wcat, bcat,
        lnt_g[None, :], lnt_b[None, :], W1.T)

    prep_kernel = _make_prep_kernel(e_pad)
    es, ed, cnt = prep_kernel(src_p, dst_p)

    scat_kernel = _make_scatter_kernel(e_pad)

    acc1_raw = scat_kernel(es, ed, cnt, y1, zeros_acc)
    acc1 = jnp.concatenate(
        [acc1_raw[:HALF], acc1_raw[NS * OUT_TPR:NS * OUT_TPR + HALF]], axis=0)

    y2 = _dense_b(acc1, y1, dinv, b1[None, :], ln1_g[None, :], ln1_b[None, :], W2.T)

    acc2_raw = scat_kernel(es, ed, cnt, y2, zeros_acc)
    acc2 = jnp.concatenate(
        [acc2_raw[:HALF], acc2_raw[NS * OUT_TPR:NS * OUT_TPR + HALF]], axis=0)

    return _dense_c(acc2, y2, dinv, b2[None, :], ln2_g[None, :], ln2_b[None, :])


# speedup vs baseline: 9.4256x; 1.1289x over previous
import functools

import jax
import jax.numpy as jnp
import numpy as np
from jax import lax
from jax.experimental import pallas as pl
from jax.experimental.pallas import tpu as pltpu
from jax.experimental.pallas import tpu_sc as plsc

N = 50000
D = 64
T = 20
NC = 2
NS = 16
LN_EPS = 1e-5

HALF = 25000
OUT_TPR = 1564
SP_TPR = 1564
SP_ROWS = NS * SP_TPR
TRASH = 25000

DEG_TPR = 3128
DEG_ROWS = NS * DEG_TPR

STREAM = 64
MEGA_E = 2560
NJ = MEGA_E // STREAM
NB = 5
NG = NJ // NB

_PERM32 = np.stack([np.arange(16), np.arange(16) + 16], axis=1).reshape(32)
_YH_PERM = np.concatenate([_PERM32, _PERM32 + 32])

CHUNK = 2048
RB_CAP = 6672
CAP_ROW = 59392


def _sc_mesh():
    return plsc.VectorSubcoreMesh(
        core_axis_name="c", subcore_axis_name="s", num_cores=NC, num_subcores=NS
    )


_SC_PARAMS = pltpu.CompilerParams(use_tc_tiling_on_sc=False)
_SC_PARAMS_NL = pltpu.CompilerParams(use_tc_tiling_on_sc=False,
                                     needs_layout_passes=False)


def _make_degree_kernel(e_pad):
    per_tile = e_pad // (NC * NS)
    n_j = per_tile // 128

    @functools.partial(
        pl.kernel,
        out_type=jax.ShapeDtypeStruct((NC * DEG_ROWS, 1), jnp.float32),
        mesh=_sc_mesh(),
        scratch_types=[
            pltpu.VMEM((per_tile,), jnp.int32),
            pltpu.VMEM((1, 128), jnp.int32),
            pltpu.VMEM((128, 1), jnp.float32),
            pltpu.VMEM((DEG_TPR, 1), jnp.float32),
            pltpu.VMEM_SHARED((DEG_ROWS, 1), jnp.float32),
        ],
        compiler_params=_SC_PARAMS,
    )
    def deg_kernel(dst_hbm, ones_hbm, zeros_hbm, out_hbm, dbuf, stg, ones_v, z_v, hist):
        cc = lax.axis_index("c")
        ss = lax.axis_index("s")
        pltpu.sync_copy(ones_hbm, ones_v)
        pltpu.sync_copy(zeros_hbm, z_v)
        pltpu.sync_copy(z_v, hist.at[pl.ds(ss * DEG_TPR, DEG_TPR)])
        plsc.subcore_barrier()

        ebase = (cc * NS + ss) * per_tile
        pltpu.sync_copy(dst_hbm.at[pl.ds(ebase, per_tile)], dbuf)

        def body(j, carry):
            def fix(u, c2):
                stg[0, pl.ds(u * 16, 16)] = dbuf[pl.ds(j * 128 + u * 16, 16)]
                return c2

            lax.fori_loop(0, 8, fix, 0)
            pltpu.sync_copy(ones_v, hist.at[stg.at[0]], add=True)
            return carry

        lax.fori_loop(0, n_j, body, 0)
        plsc.subcore_barrier()
        pltpu.sync_copy(
            hist.at[pl.ds(ss * DEG_TPR, DEG_TPR)],
            out_hbm.at[pl.ds(cc * DEG_ROWS + ss * DEG_TPR, DEG_TPR)],
        )

    return deg_kernel


def _make_prep_kernel(e_pad):
    per_tile = e_pad // NS
    n_chunks = per_tile // CHUNK

    @functools.partial(
        pl.kernel,
        out_type=[
            jax.ShapeDtypeStruct((NC * NS * CAP_ROW,), jnp.int32),
            jax.ShapeDtypeStruct((NC * NS * CAP_ROW,), jnp.int32),
            jax.ShapeDtypeStruct((NC * NS, 16), jnp.int32),
        ],
        mesh=_sc_mesh(),
        scratch_types=[
            pltpu.VMEM((CHUNK,), jnp.int32),
            pltpu.VMEM((CHUNK,), jnp.int32),
            pltpu.VMEM((RB_CAP,), jnp.int32),
            pltpu.VMEM((RB_CAP,), jnp.int32),
            pltpu.VMEM((1, 16), jnp.int32),
        ],
        compiler_params=_SC_PARAMS_NL,
    )
    def prep_kernel(src_hbm, dst_hbm, es_hbm, ed_hbm, cnt_hbm,
                    in_s, in_d, rb_s, rb_d, cbuf):
        cc = lax.axis_index("c")
        ss = lax.axis_index("s")
        base = cc * HALF
        out0 = (cc * NS + ss) * CAP_ROW

        def flush_block(off):
            off = pl.multiple_of(off, CHUNK)
            pltpu.sync_copy(rb_s.at[pl.ds(0, CHUNK)],
                            es_hbm.at[pl.ds(out0 + off, CHUNK)])
            pltpu.sync_copy(rb_d.at[pl.ds(0, CHUNK)],
                            ed_hbm.at[pl.ds(out0 + off, CHUNK)])

            def mv(i, c4):
                rb_s[pl.ds(i * 16, 16)] = rb_s[pl.ds(CHUNK + i * 16, 16)]
                rb_d[pl.ds(i * 16, 16)] = rb_d[pl.ds(CHUNK + i * 16, 16)]
                return c4

            lax.fori_loop(0, (RB_CAP - CHUNK) // 16, mv, 0)

        lane15 = jnp.full((16,), 15, jnp.int32)
        iota16 = lax.iota(jnp.int32, 16)

        def chunk(ci, carry):
            pend_v, off = carry
            ebase = ss * per_tile + ci * CHUNK
            pltpu.sync_copy(src_hbm.at[pl.ds(ebase, CHUNK)], in_s)
            pltpu.sync_copy(dst_hbm.at[pl.ds(ebase, CHUNK)], in_d)

            def vec(v, pend2):
                sv = in_s[pl.ds(v * 16, 16)]
                dv = in_d[pl.ds(v * 16, 16)]
                loc = dv - base
                ok = (loc >= 0) & (loc < HALF)
                okv = jnp.where(ok, 1, 0).astype(jnp.int32)
                incl = plsc.cumsum(okv)
                pos = pend2 + incl - okv
                plsc.store_scatter(rb_s, [pos], sv, mask=ok)
                plsc.store_scatter(rb_d, [pos], loc, mask=ok)
                return pend2 + incl[lane15]

            pend_v = lax.fori_loop(0, CHUNK // 16, vec, pend_v)
            pend_s = lax.reduce_max(pend_v, (0,))

            def do_flush(args):
                pv3, o3 = args
                flush_block(o3)
                return pv3 - CHUNK, o3 + CHUNK

            pend_v, off = lax.cond(pend_s >= CHUNK, do_flush, lambda a: a,
                                   (pend_v, off))
            return pend_v, off

        pend_v, off = lax.fori_loop(
            0, n_chunks, chunk, (jnp.zeros((16,), jnp.int32), 0))
        pend = lax.reduce_max(pend_v, (0,))
        cnt = off + pend

        def padv(i, c2):
            idxv = pend_v + i * 16 + iota16
            plsc.store_scatter(rb_s, [idxv], jnp.zeros((16,), jnp.int32))
            plsc.store_scatter(rb_d, [idxv], jnp.full((16,), TRASH, jnp.int32))
            return c2

        lax.fori_loop(0, MEGA_E // 16, padv, 0)

        def fin(k, off5):
            flush_block(off5)
            return off5 + CHUNK

        lax.fori_loop(0, 3, fin, off)

        cbuf[0, pl.ds(0, 16)] = jnp.full((16,), 1, jnp.int32) * cnt
        pltpu.sync_copy(cbuf, cnt_hbm.at[pl.ds(cc * NS + ss, 1)])

    return prep_kernel


def _make_scatter_kernel(e_pad):
    per_tile = e_pad // NS
    n_megas = per_tile // MEGA_E

    @functools.partial(
        pl.kernel,
        out_type=jax.ShapeDtypeStruct((NC * NS * OUT_TPR, D), jnp.float32),
        mesh=_sc_mesh(),
        scratch_types=[
            pltpu.VMEM((MEGA_E,), jnp.int32),
            pltpu.VMEM((MEGA_E,), jnp.int32),
            pltpu.VMEM((2, STREAM), jnp.int32),
            pltpu.VMEM((NB, STREAM, D), jnp.bfloat16),
            pltpu.VMEM((2, STREAM, D), jnp.float32),
            pltpu.VMEM((1, 16), jnp.int32),
            pltpu.SemaphoreType.DMA,
            pltpu.SemaphoreType.DMA,
            pltpu.VMEM_SHARED((SP_ROWS, D), jnp.float32),
        ],
        compiler_params=_SC_PARAMS_NL,
    )
    def scat_kernel(es_hbm, ed_hbm, cnt_hbm, yh_hbm, zeros_hbm, out_hbm,
                    sbuf, dbuf, stg, rows_h, rows_f, cbuf, gsem, ssem, acc):
        cc = lax.axis_index("c")
        ss = lax.axis_index("s")
        pltpu.sync_copy(zeros_hbm, acc.at[pl.ds(ss * SP_TPR, SP_TPR)])
        pltpu.sync_copy(cnt_hbm.at[pl.ds(cc * NS + ss, 1)], cbuf)
        plsc.subcore_barrier()

        out0 = (cc * NS + ss) * CAP_ROW
        cnt = lax.reduce_max(cbuf[0, pl.ds(0, 16)], (0,))
        nm = (cnt + MEGA_E - 1) // MEGA_E
        himask = jnp.full((16,), -65536, jnp.int32)

        def fix(jj, fb):
            def f(u, c3):
                stg[fb, pl.ds(u * 16, 16)] = dbuf[pl.ds(jj * STREAM + u * 16, 16)]
                return c3

            lax.fori_loop(0, STREAM // 16, f, 0)

        def convert(bh, fb):
            def f(r, c3):
                for c in range(2):
                    x = rows_h[bh, r, pl.ds(32 * c, 32)]
                    xi = plsc.bitcast(x, jnp.int32)
                    lo = plsc.bitcast(lax.shift_left(xi, 16), jnp.float32)
                    hi = plsc.bitcast(xi & himask, jnp.float32)
                    rows_f[fb, r, pl.ds(32 * c, 16)] = lo
                    rows_f[fb, r, pl.ds(32 * c + 16, 16)] = hi
                return c3

            lax.fori_loop(0, STREAM, f, 0)

        def issue_gather(jj, bh):
            pltpu.async_copy(yh_hbm.at[sbuf.at[pl.ds(jj * STREAM, STREAM)]],
                             rows_h.at[bh], gsem)

        def wait_gather(bh):
            pltpu.make_async_copy(yh_hbm.at[pl.ds(0, STREAM)], rows_h.at[bh],
                                  gsem).wait()

        def issue_scatter(fb):
            pltpu.async_copy(rows_f.at[fb], acc.at[stg.at[fb]], ssem, add=True)

        def drain_scatter():
            pltpu.make_async_copy(zeros_hbm.at[pl.ds(0, STREAM)], rows_f.at[0],
                                  ssem).wait()

        def mega(m, carry):
            ebase = out0 + m * MEGA_E
            pltpu.sync_copy(es_hbm.at[pl.ds(ebase, MEGA_E)], sbuf)
            pltpu.sync_copy(ed_hbm.at[pl.ds(ebase, MEGA_E)], dbuf)
            for b in range(NB - 1):
                issue_gather(b, b)

            def grp(g, c2):
                for u in range(2 * NB):
                    j = g * 2 * NB + u
                    bh = u % NB
                    fb = u % 2
                    wait_gather(bh)

                    @pl.when(j >= 2)
                    def _():
                        drain_scatter()

                    fix(j, fb)
                    convert(bh, fb)
                    issue_scatter(fb)

                    @pl.when(j + NB - 1 < NJ)
                    def _():
                        issue_gather(j + NB - 1, (u + NB - 1) % NB)
                return c2

            lax.fori_loop(0, NJ // (2 * NB), grp, 0)
            drain_scatter()
            drain_scatter()
            return carry

        lax.fori_loop(0, nm, mega, 0)
        plsc.subcore_barrier()
        pltpu.sync_copy(
            acc.at[pl.ds(ss * OUT_TPR, OUT_TPR)],
            out_hbm.at[pl.ds((cc * NS + ss) * OUT_TPR, OUT_TPR)],
        )

    return scat_kernel


BN = 2000


def _ln_rows(x, g, b):
    m = jnp.mean(x, axis=-1, keepdims=True)
    v = jnp.mean((x - m) ** 2, axis=-1, keepdims=True)
    return (x - m) / jnp.sqrt(v + LN_EPS) * g + b


def _dense_a_body(bw_ref, gu_ref, mu_ref, gf_ref, mc_ref, rid_ref, d0_ref, d1_ref,
                  remb_ref, wgf_ref, wmc_ref, bs_ref, wcat_ref, bcat_ref,
                  lng_ref, lnb_ref, w1t_ref,
                  y1_ref, dinv_ref):
    gf = jnp.log1p(jnp.maximum(gf_ref[...], 0.0))
    mc = jnp.log1p(jnp.maximum(mc_ref[...], 0.0))
    rid = rid_ref[...]
    h_static = gf * wgf_ref[...] + mc * wmc_ref[...] + bs_ref[...]
    for k in range(3):
        h_static += jnp.where(rid == k, 1.0, 0.0) * remb_ref[k:k + 1, :]

    bw = jnp.log1p(jnp.maximum(bw_ref[...], 0.0))
    gu = gu_ref[...]
    mu = mu_ref[...]
    wcat = wcat_ref[...]
    bcat = bcat_ref[...]

    h = jnp.zeros((BN, D), jnp.float32)
    for t in range(T):
        x_cat = jnp.concatenate(
            [bw[:, t:t + 1], gu[:, t:t + 1], mu[:, t:t + 1], h], axis=-1)
        g = jnp.dot(x_cat, wcat, preferred_element_type=jnp.float32) + bcat
        rz = jax.nn.sigmoid(g[:, :2 * D])
        r = rz[:, :D]
        z = rz[:, D:2 * D]
        n = jnp.tanh(g[:, 3 * D:] + r * g[:, 2 * D:3 * D])
        h = n + z * (h - n)

    h_dyn = _ln_rows(h, lng_ref[...], lnb_ref[...])
    x0 = jnp.maximum(h_static + h_dyn, 0.0)
    dinv = 1.0 / jnp.sqrt(d0_ref[...] + d1_ref[...] + 1.0)
    y1_ref[...] = jnp.dot(x0, w1t_ref[...], preferred_element_type=jnp.float32) * dinv
    dinv_ref[...] = dinv


def _dense_a(bw, gu, mu, gf, mc, rid, d0, d1, remb, wgf, wmc, bs_row,
             wcat, bcat, lng, lnb, w1t):
    nb = N // BN
    row = lambda i: (i, 0)
    full = lambda i: (0, 0)
    spec = lambda shape, im: pl.BlockSpec(shape, im)
    return pl.pallas_call(
        _dense_a_body,
        grid=(nb,),
        in_specs=[
            spec((BN, T), row), spec((BN, T), row), spec((BN, T), row),
            spec((BN, 1), row), spec((BN, 1), row), spec((BN, 1), row),
            spec((BN, 1), row), spec((BN, 1), row),
            spec((3, D), full), spec((1, D), full), spec((1, D), full),
            spec((1, D), full), spec((3 + D, 4 * D), full), spec((1, 4 * D), full),
            spec((1, D), full), spec((1, D), full), spec((D, D), full),
        ],
        out_specs=[spec((BN, D), row), spec((BN, 1), row)],
        out_shape=[
            jax.ShapeDtypeStruct((N, D), jnp.float32),
            jax.ShapeDtypeStruct((N, 1), jnp.float32),
        ],
    )(bw, gu, mu, gf, mc, rid, d0, d1, remb, wgf, wmc, bs_row,
      wcat, bcat, lng, lnb, w1t)


def _dense_b_body(acc_ref, y_ref, dinv_ref, b_ref, lng_ref, lnb_ref, w2t_ref, out_ref):
    dinv = dinv_ref[...]
    o = dinv * (acc_ref[...] + y_ref[...]) + b_ref[...]
    h1 = _ln_rows(jnp.maximum(o, 0.0), lng_ref[...], lnb_ref[...])
    out_ref[...] = jnp.dot(h1, w2t_ref[...], preferred_element_type=jnp.float32) * dinv


def _dense_b(acc1, y1, dinv, b_row, lng, lnb, w2t):
    nb = N // BN
    row = lambda i: (i, 0)
    full = lambda i: (0, 0)
    spec = lambda shape, im: pl.BlockSpec(shape, im)
    return pl.pallas_call(
        _dense_b_body,
        grid=(nb,),
        in_specs=[
            spec((BN, D), row), spec((BN, D), row), spec((BN, 1), row),
            spec((1, D), full), spec((1, D), full), spec((1, D), full),
            spec((D, D), full),
        ],
        out_specs=spec((BN, D), row),
        out_shape=jax.ShapeDtypeStruct((N, D), jnp.float32),
    )(acc1, y1, dinv, b_row, lng, lnb, w2t)


def _dense_c_body(acc_ref, y_ref, dinv_ref, b_ref, lng_ref, lnb_ref, out_ref):
    o = dinv_ref[...] * (acc_ref[...] + y_ref[...]) + b_ref[...]
    out_ref[...] = _ln_rows(o, lng_ref[...], lnb_ref[...])


def _dense_c(acc2, y2, dinv, b_row, lng, lnb):
    nb = N // BN
    row = lambda i: (i, 0)
    full = lambda i: (0, 0)
    spec = lambda shape, im: pl.BlockSpec(shape, im)
    return pl.pallas_call(
        _dense_c_body,
        grid=(nb,),
        in_specs=[
            spec((BN, D), row), spec((BN, D), row), spec((BN, 1), row),
            spec((1, D), full), spec((1, D), full), spec((1, D), full),
        ],
        out_specs=spec((BN, D), row),
        out_shape=jax.ShapeDtypeStruct((N, D), jnp.float32),
    )(acc2, y2, dinv, b_row, lng, lnb)


def kernel(edge_index, gpu_flops, role_id, mem_capacity, bandwidth_seq,
           gpu_util_seq, mem_util_seq, role_emb, Ws, bs, Wih, Whh, bih, bhh,
           lnt_g, lnt_b, W1, b1, W2, b2, ln1_g, ln1_b, ln2_g, ln2_b):
    E = edge_index.shape[1]
    grain = NS * MEGA_E
    e_pad = ((E + grain - 1) // grain) * grain

    src = edge_index[0]
    dst = edge_index[1]
    src_p = jnp.concatenate([src, jnp.zeros((e_pad - E,), jnp.int32)])
    dst_p = jnp.concatenate([dst, jnp.full((e_pad - E,), N, jnp.int32)])

    ones_col = jnp.ones((128, 1), jnp.float32)
    zeros_deg = jnp.zeros((DEG_TPR, 1), jnp.float32)
    zeros_acc = jnp.zeros((SP_TPR, D), jnp.float32)

    deg_kernel = _make_degree_kernel(e_pad)
    deg_parts = deg_kernel(dst_p, ones_col, zeros_deg)
    d0 = deg_parts[:N]
    d1 = deg_parts[DEG_ROWS:DEG_ROWS + N]

    WsT = Ws.T
    wgf = WsT[0:1, :]
    wmc = WsT[1:2, :]
    remb = role_emb @ WsT[2:10, :]
    bs_row = bs[None, :]
    wihT = Wih.T
    whhT = Whh.T
    z3 = jnp.zeros((3, D), jnp.float32)
    zD = jnp.zeros((D, D), jnp.float32)
    wcat = jnp.concatenate([
        jnp.concatenate([wihT[:, :D], wihT[:, D:2 * D], z3, wihT[:, 2 * D:]], 1),
        jnp.concatenate([whhT[:, :D], whhT[:, D:2 * D], whhT[:, 2 * D:], zD], 1),
    ], axis=0)
    bcat = jnp.concatenate(
        [bih[:D] + bhh[:D], bih[D:2 * D] + bhh[D:2 * D],
         bhh[2 * D:], bih[2 * D:]])[None, :]

    y1, dinv = _dense_a(
        bandwidth_seq, gpu_util_seq, mem_util_seq,
        gpu_flops[:, None], mem_capacity[:, None], role_id[:, None],
        d0, d1, remb, wgf, wmc, bs_row, wcat, bcat,
        lnt_g[None, :], lnt_b[None, :], W1.T)

    prep_kernel = _make_prep_kernel(e_pad)
    es, ed, cnt = prep_kernel(src_p, dst_p)

    scat_kernel = _make_scatter_kernel(e_pad)

    y1h = y1[:, _YH_PERM].astype(jnp.bfloat16)
    acc1_raw = scat_kernel(es, ed, cnt, y1h, zeros_acc)
    acc1 = jnp.concatenate(
        [acc1_raw[:HALF], acc1_raw[NS * OUT_TPR:NS * OUT_TPR + HALF]], axis=0)

    y2 = _dense_b(acc1, y1, dinv, b1[None, :], ln1_g[None, :], ln1_b[None, :], W2.T)

    y2h = y2[:, _YH_PERM].astype(jnp.bfloat16)
    acc2_raw = scat_kernel(es, ed, cnt, y2h, zeros_acc)
    acc2 = jnp.concatenate(
        [acc2_raw[:HALF], acc2_raw[NS * OUT_TPR:NS * OUT_TPR + HALF]], axis=0)

    return _dense_c(acc2, y2, dinv, b2[None, :], ln2_g[None, :], ln2_b[None, :])
